# Initial kernel scaffold; baseline (speedup 1.0000x reference)
import functools

import jax
import jax.numpy as jnp
from jax import lax
from jax.experimental import pallas as pl
from jax.experimental.pallas import tpu as pltpu
from jax.experimental.pallas import tpu_sc as plsc

NC = 2
NS = 16
NW = NC * NS
CW = 128
D = 64
CHUNKS = 50
NBUF = 5
B = NW * CHUNKS * CW

_mesh = plsc.VectorSubcoreMesh(
    core_axis_name="c", subcore_axis_name="s", num_cores=NC, num_subcores=NS
)


@functools.partial(
    pl.kernel,
    out_type=jax.ShapeDtypeStruct((B, D), jnp.float32),
    mesh=_mesh,
    scratch_types=(
        [pltpu.VMEM((CHUNKS, CW), jnp.int32)]
        + [pltpu.VMEM((NBUF, CW, D), jnp.float32)]
        + [pltpu.SemaphoreType.DMA] * (1 + 2 * NBUF)
    ),
)
def _emb_lookup(table_hbm, idx_hbm, out_hbm, idx_v, rows_v, isem, *sems):
    gsems = sems[:NBUF]
    ssems = sems[NBUF:]
    wid = lax.axis_index("s") * NC + lax.axis_index("c")

    pltpu.async_copy(idx_hbm.at[pl.ds(wid * CHUNKS, CHUNKS)], idx_v, isem).wait()

    base_row = wid * CHUNKS * CW

    def start_gather(g, b):
        pltpu.async_copy(table_hbm.at[idx_v.at[g]], rows_v.at[b], gsems[b])

    def wait_gather(b):
        pltpu.make_async_copy(
            out_hbm.at[pl.ds(0, CW)], rows_v.at[b], gsems[b]
        ).wait()

    def start_scatter(g, b):
        pltpu.async_copy(
            rows_v.at[b], out_hbm.at[pl.ds(base_row + g * CW, CW)], ssems[b]
        )

    def wait_scatter(b):
        pltpu.make_async_copy(
            rows_v.at[b], out_hbm.at[pl.ds(0, CW)], ssems[b]
        ).wait()

    for b in range(NBUF):
        start_gather(b, b)

    def body(i, carry):
        for b in range(NBUF):
            g = i * NBUF + b
            wait_gather(b)
            start_scatter(g, b)
            wait_scatter(b)
            start_gather(g + NBUF, b)
        return carry

    lax.fori_loop(0, CHUNKS // NBUF - 1, body, 0)

    for b in range(NBUF):
        g = (CHUNKS - NBUF) + b
        wait_gather(b)
        start_scatter(g, b)
    for b in range(NBUF):
        wait_scatter(b)


def kernel(x, table):
    if x.ndim > 1 and x.shape[-1] == 1:
        x = x[..., 0]
    lead_shape = x.shape
    idx = x.reshape(NW * CHUNKS, CW).astype(jnp.int32)
    out = _emb_lookup(table.astype(jnp.float32), idx)
    return out.reshape(*lead_shape, D)


# speedup vs baseline: 3.1360x; 3.1360x over previous
import functools

import jax
import jax.numpy as jnp
from jax import lax
from jax.experimental import pallas as pl
from jax.experimental.pallas import tpu as pltpu
from jax.experimental.pallas import tpu_sc as plsc

NC = 2
NS = 16
NW = NC * NS
CW = 128
D = 64
CHUNKS = 50
NBUF = 5
B = NW * CHUNKS * CW

_mesh = plsc.VectorSubcoreMesh(
    core_axis_name="c", subcore_axis_name="s", num_cores=NC, num_subcores=NS
)


@functools.partial(
    pl.kernel,
    out_type=jax.ShapeDtypeStruct((B, D), jnp.float32),
    mesh=_mesh,
    scratch_types=(
        [pltpu.VMEM((CHUNKS, CW), jnp.int32)]
        + [pltpu.VMEM((NBUF, CW, D), jnp.float32)]
        + [pltpu.SemaphoreType.DMA] * (1 + 2 * NBUF)
    ),
    compiler_params=pltpu.CompilerParams(use_tc_tiling_on_sc=False),
)
def _emb_lookup(table_hbm, idx_hbm, out_hbm, idx_v, rows_v, isem, *sems):
    gsems = sems[:NBUF]
    ssems = sems[NBUF:]
    wid = lax.axis_index("s") * NC + lax.axis_index("c")

    pltpu.async_copy(idx_hbm.at[wid], idx_v, isem).wait()

    base_row = wid * CHUNKS * CW

    def start_gather(g, b):
        pltpu.async_copy(table_hbm.at[idx_v.at[g]], rows_v.at[b], gsems[b])

    def wait_gather(b):
        pltpu.make_async_copy(
            out_hbm.at[pl.ds(0, CW)], rows_v.at[b], gsems[b]
        ).wait()

    def start_scatter(g, b):
        pltpu.async_copy(
            rows_v.at[b], out_hbm.at[pl.ds(base_row + g * CW, CW)], ssems[b]
        )

    def wait_scatter(b):
        pltpu.make_async_copy(
            rows_v.at[b], out_hbm.at[pl.ds(0, CW)], ssems[b]
        ).wait()

    for b in range(NBUF):
        start_gather(b, b)

    def body(i, carry):
        for b in range(NBUF):
            g = i * NBUF + b
            wait_gather(b)
            start_scatter(g, b)
            wait_scatter(b)
            start_gather(g + NBUF, b)
        return carry

    lax.fori_loop(0, CHUNKS // NBUF - 1, body, 0)

    for b in range(NBUF):
        g = (CHUNKS - NBUF) + b
        wait_gather(b)
        start_scatter(g, b)
    for b in range(NBUF):
        wait_scatter(b)


def kernel(x, table):
    if x.ndim > 1 and x.shape[-1] == 1:
        x = x[..., 0]
    lead_shape = x.shape
    idx = x.reshape(NW, CHUNKS, CW).astype(jnp.int32)
    out = _emb_lookup(table.astype(jnp.float32), idx)
    return out.reshape(*lead_shape, D)
